# baseline (speedup 1.0000x reference)
import functools

import jax
import jax.numpy as jnp
from jax import lax
from jax.experimental import pallas as pl
from jax.experimental.pallas import tpu as pltpu
from jax.experimental.pallas import tpu_sc as plsc

_L = 16
_NW = 32


def _ffloor(x, vone):
    t = x.astype(jnp.int32).astype(jnp.float32)
    return jnp.where(t > x, t - vone, t)


def _fceil(x, vone):
    t = x.astype(jnp.int32).astype(jnp.float32)
    return jnp.where(t < x, t + vone, t)


def _split(n):
    for w in range(_NW, 0, -1):
        if n % w == 0 and (n // w) % _L == 0:
            return w, n // w
    raise ValueError(f"cannot split {n} boxes across subcores")


@functools.partial(jax.jit, static_argnums=(1, 2, 3, 4, 5))
def _run(bbs, im_h, im_w, enlargement_factor, target_size, min_len):
    n = bbs.shape[0]
    n_workers, b_per_w = _split(n)
    nvec = b_per_w // _L

    mesh = plsc.VectorSubcoreMesh(core_axis_name="c", subcore_axis_name="s")

    @functools.partial(
        pl.kernel,
        mesh=mesh,
        out_type=[jax.ShapeDtypeStruct((n,), jnp.float32)] * 6,
        scratch_types=[pltpu.VMEM((b_per_w,), jnp.float32)] * 10,
    )
    def run(xs_h, ys_h, ws_h, hs_h, ox_h, oy_h, ow_h, oh_h, otw_h, oth_h,
            v_x, v_y, v_w, v_h, v_ix, v_iy, v_iw, v_ih, v_tw, v_th):
        wid = lax.axis_index("s") * 2 + lax.axis_index("c")

        @pl.when(wid < n_workers)
        def _():
            base = wid * b_per_w
            sl = pl.ds(base, b_per_w)
            pltpu.sync_copy(xs_h.at[sl], v_x)
            pltpu.sync_copy(ys_h.at[sl], v_y)
            pltpu.sync_copy(ws_h.at[sl], v_w)
            pltpu.sync_copy(hs_h.at[sl], v_h)

            f32 = jnp.float32
            vec = lambda v: jnp.full((_L,), v, f32)
            ef = f32(enlargement_factor)
            half = f32(0.5)
            vone = vec(1.0)
            vzero = vec(0.0)
            vts = vec(target_size)
            vml = vec(min_len)
            vfh = vec(im_h)
            vfw = vec(im_w)
            vfwml = vec(im_w - min_len)
            vfhml = vec(im_h - min_len)

            def body(i, carry):
                vs = pl.ds(i * _L, _L)
                bx = v_x[vs]
                by = v_y[vs]
                bw = v_w[vs]
                bh = v_h[vs]

                w = _fceil(bw * ef, vone)
                h = _fceil(bh * ef, vone)
                ix = _ffloor(bx - w * half, vone)
                c = ix < vzero
                w = jnp.where(c, w + ix, w)
                ix = jnp.where(c, vzero, ix)
                iy = _ffloor(by - h * half, vone)
                c = iy < vzero
                h = jnp.where(c, h + iy, h)
                iy = jnp.where(c, vzero, iy)
                w = jnp.maximum(w, vml)
                h = jnp.maximum(h, vml)
                iw = vfw - ix
                iw = jnp.where(w < iw, w, iw)
                ih = vfh - iy
                ih = jnp.where(h < ih, h, ih)
                idx = iw < vml
                iw = jnp.where(idx, vml, iw)
                ix = jnp.where(idx, vfwml, ix)
                idx = ih < vml
                ih = jnp.where(idx, vml, ih)
                iy = jnp.where(idx, vfhml, iy)
                tw = jnp.where(iw > ih, vts * iw / ih, vts)
                th = jnp.where(iw <= ih, vts * ih / iw, vts)

                sl = pl.ds(i * _L, _L)
                v_ix[sl] = ix
                v_iy[sl] = iy
                v_iw[sl] = iw
                v_ih[sl] = ih
                v_tw[sl] = tw
                v_th[sl] = th
                return carry

            lax.fori_loop(0, nvec, body, 0)

            sl = pl.ds(base, b_per_w)
            pltpu.sync_copy(v_ix, ox_h.at[sl])
            pltpu.sync_copy(v_iy, oy_h.at[sl])
            pltpu.sync_copy(v_iw, ow_h.at[sl])
            pltpu.sync_copy(v_ih, oh_h.at[sl])
            pltpu.sync_copy(v_tw, otw_h.at[sl])
            pltpu.sync_copy(v_th, oth_h.at[sl])

    return run(bbs[:, 0], bbs[:, 1], bbs[:, 2], bbs[:, 3])


def kernel(img, bbs):
    im_h = float(img.shape[2])
    im_w = float(img.shape[3])
    out = _run(bbs, im_h, im_w, 1.5, 256, 3.0)
    return tuple(out)


# speedup vs baseline: 1.0813x; 1.0813x over previous
import functools

import jax
import jax.numpy as jnp
from jax import lax
from jax.experimental import pallas as pl
from jax.experimental.pallas import tpu as pltpu
from jax.experimental.pallas import tpu_sc as plsc

_L = 16
_NW = 32


def _ffloor(x, vone):
    t = x.astype(jnp.int32).astype(jnp.float32)
    return jnp.where(t > x, t - vone, t)


def _fceil(x, vone):
    t = x.astype(jnp.int32).astype(jnp.float32)
    return jnp.where(t < x, t + vone, t)


def _chunk(n):
    step = 8 * _L
    c = -(-n // _NW)
    c = -(-c // step) * step
    if c > n:
        raise ValueError(f"n={n} too small for {_NW} workers")
    return c


@functools.partial(jax.jit, static_argnums=(1, 2, 3, 4, 5))
def _run(bbs, im_h, im_w, enlargement_factor, target_size, min_len):
    n = bbs.shape[0]
    c = _chunk(n)
    nvec = c // _L

    mesh = plsc.VectorSubcoreMesh(core_axis_name="c", subcore_axis_name="s")

    @functools.partial(
        pl.kernel,
        mesh=mesh,
        out_type=[jax.ShapeDtypeStruct((n,), jnp.float32)] * 6,
        scratch_types=(
            [pltpu.VMEM((c,), jnp.float32)] * 10
            + [pltpu.SemaphoreType.DMA] * 2
        ),
    )
    def run(xs_h, ys_h, ws_h, hs_h, ox_h, oy_h, ow_h, oh_h, otw_h, oth_h,
            v_x, v_y, v_w, v_h,
            v_ix, v_iy, v_iw, v_ih, v_tw, v_th, sem_in, sem_out):
        wid = lax.axis_index("s") * 2 + lax.axis_index("c")
        base = jnp.minimum(wid * c, n - c)
        isl = pl.ds(base, c)
        in_cps = [
            pltpu.async_copy(xs_h.at[isl], v_x, sem_in),
            pltpu.async_copy(ys_h.at[isl], v_y, sem_in),
            pltpu.async_copy(ws_h.at[isl], v_w, sem_in),
            pltpu.async_copy(hs_h.at[isl], v_h, sem_in),
        ]

        f32 = jnp.float32
        vec = lambda v: jnp.full((_L,), v, f32)
        ef = f32(enlargement_factor)
        half = f32(0.5)
        vone = vec(1.0)
        vzero = vec(0.0)
        vts = vec(target_size)
        vml = vec(min_len)
        vfh = vec(im_h)
        vfw = vec(im_w)
        vfwml = vec(im_w - min_len)
        vfhml = vec(im_h - min_len)

        for cp in in_cps:
            cp.wait()

        def body(i, carry):
            vs = pl.ds(i * _L, _L)
            bx = v_x[vs]
            by = v_y[vs]
            bw = v_w[vs]
            bh = v_h[vs]

            w = _fceil(bw * ef, vone)
            h = _fceil(bh * ef, vone)
            ix = _ffloor(bx - w * half, vone)
            cnd = ix < vzero
            w = jnp.where(cnd, w + ix, w)
            ix = jnp.where(cnd, vzero, ix)
            iy = _ffloor(by - h * half, vone)
            cnd = iy < vzero
            h = jnp.where(cnd, h + iy, h)
            iy = jnp.where(cnd, vzero, iy)
            w = jnp.maximum(w, vml)
            h = jnp.maximum(h, vml)
            iw = vfw - ix
            iw = jnp.where(w < iw, w, iw)
            ih = vfh - iy
            ih = jnp.where(h < ih, h, ih)
            idx = iw < vml
            iw = jnp.where(idx, vml, iw)
            ix = jnp.where(idx, vfwml, ix)
            idx = ih < vml
            ih = jnp.where(idx, vml, ih)
            iy = jnp.where(idx, vfhml, iy)
            tw = jnp.where(iw > ih, vts * iw / ih, vts)
            th = jnp.where(iw <= ih, vts * ih / iw, vts)

            sl = pl.ds(i * _L, _L)
            v_ix[sl] = ix
            v_iy[sl] = iy
            v_iw[sl] = iw
            v_ih[sl] = ih
            v_tw[sl] = tw
            v_th[sl] = th
            return carry

        lax.fori_loop(0, nvec, body, 0)

        sl = pl.ds(base, c)
        cps = [
            pltpu.async_copy(v_ix, ox_h.at[sl], sem_out),
            pltpu.async_copy(v_iy, oy_h.at[sl], sem_out),
            pltpu.async_copy(v_iw, ow_h.at[sl], sem_out),
            pltpu.async_copy(v_ih, oh_h.at[sl], sem_out),
            pltpu.async_copy(v_tw, otw_h.at[sl], sem_out),
            pltpu.async_copy(v_th, oth_h.at[sl], sem_out),
        ]
        for cp in cps:
            cp.wait()

    return run(bbs[:, 0], bbs[:, 1], bbs[:, 2], bbs[:, 3])


def kernel(img, bbs):
    im_h = float(img.shape[2])
    im_w = float(img.shape[3])
    out = _run(bbs, im_h, im_w, 1.5, 256, 3.0)
    return tuple(out)
